# baseline (speedup 1.0000x reference)
import functools

import jax
import jax.numpy as jnp
from jax import lax
from jax.experimental import pallas as pl
from jax.experimental.pallas import tpu as pltpu
from jax.experimental.pallas import tpu_sc as plsc

_PATCH = 14
_NPOS = _PATCH * _PATCH
_NPOSP = 208
_NPAD = 200
_NLEV = 256
_NWORDS = _NPAD * _NLEV
_IBT = 32


def _sc_hist_body(x_hbm, n_hbm, xv, nbuf, nbuf2, sem0, sem1, imgs_per_worker, num_cores):
    wid = lax.axis_index("s") * num_cores + lax.axis_index("c")
    zeros = jnp.zeros((16,), jnp.float32)
    ones = jnp.ones((16,), jnp.float32)
    iota16 = lax.broadcasted_iota(jnp.int32, (16,), 0)

    def _clear(r, carry):
        for k in range(_NLEV // 16):
            nbuf[r, pl.ds(k * 16, 16)] = zeros
            nbuf2[r, pl.ds(k * 16, 16)] = zeros
        return carry

    lax.fori_loop(0, _NPAD, _clear, 0)

    def _indices(img, chunk):
        p, c = chunk
        xc = xv[pl.ds(img * 4 * _NPOSP + p * _NPOSP + c * 16, 16)]
        t = xc * float(_NLEV - 1) + 8388608.0
        lv = (t - 8388608.0).astype(jnp.int32)
        lv = jnp.minimum(jnp.maximum(lv, 0), _NLEV - 1)
        jv = jnp.minimum(iota16 + c * 16, _NPAD - 1)
        mask = (iota16 + c * 16) < (_NPOS if c == 12 else _NPOSP)
        return jv, lv, mask

    chunks = [(p, c) for p in range(4) for c in range(13)]
    base = wid * imgs_per_worker
    pltpu.sync_copy(x_hbm.at[pl.ds(base * 4 * _NPOSP, imgs_per_worker * 4 * _NPOSP)], xv)
    bufs = (nbuf, nbuf2)
    sems = (sem0, sem1)
    copies = [None, None]
    for img in range(imgs_per_worker):
        k = img % 2
        buf = bufs[k]
        if copies[k] is not None:
            copies[k].wait()
            for ch in chunks:
                jv, lv, mask = _indices(img - 2, ch)
                plsc.store_scatter(buf, [jv, lv], zeros, mask=mask)
        for ch in chunks:
            jv, lv, mask = _indices(img, ch)
            plsc.addupdate_scatter(buf, [jv, lv], ones, mask=mask)
        copies[k] = pltpu.async_copy(
            buf, n_hbm.at[pl.ds((base + img) * _NPAD, _NPAD)], sems[k])
    for k in range(2):
        if copies[k] is not None:
            copies[k].wait()


def _tc_body(n_ref, pw_ref, lw_ref, out_ref):
    pw = pw_ref[...]
    lw = lw_ref[...]
    for i in range(_IBT):
        cnt = n_ref[i * _NPAD:(i + 1) * _NPAD, :].astype(jnp.bfloat16)
        m = jax.lax.dot_general(
            cnt, lw, (((1,), (0,)), ((), ())),
            preferred_element_type=jnp.float32,
        )
        s = jnp.sum(m * pw, axis=0)
        out_ref[i, :] = jnp.where(s > 0.0, 1.0, -1.0)


def kernel(x, position_weight, level_weight):
    B, C, H, W = x.shape
    p = _PATCH
    D = position_weight.shape[1]
    x_pj = x.reshape(B, C, H // p, p, W // p, p)
    x_pj = x_pj.transpose(0, 1, 2, 4, 3, 5).reshape(B, 4, p * p)
    x_pj = jnp.pad(x_pj, ((0, 0), (0, 0), (0, _NPOSP - _NPOS)))
    x_sc = x_pj.reshape(B * 4 * _NPOSP)

    info = plsc.get_sparse_core_info()
    nw = info.num_cores * info.num_subcores
    sc_hist = functools.partial(
        _sc_hist_body,
        imgs_per_worker=B // 2 // nw,
        num_cores=info.num_cores,
    )
    Bh = B // 2
    sc_call = pl.kernel(
        sc_hist,
        out_type=jax.ShapeDtypeStruct((Bh * _NPAD, _NLEV), jnp.float32),
        mesh=plsc.VectorSubcoreMesh(
            core_axis_name="c", subcore_axis_name="s"),
        compiler_params=pltpu.CompilerParams(needs_layout_passes=False),
        scratch_types=[
            pltpu.VMEM((Bh // nw * 4 * _NPOSP,), jnp.float32),
            pltpu.VMEM((_NPAD, _NLEV), jnp.float32),
            pltpu.VMEM((_NPAD, _NLEV), jnp.float32),
            pltpu.SemaphoreType.DMA,
            pltpu.SemaphoreType.DMA,
        ],
    )
    half = Bh * 4 * _NPOSP
    n0 = sc_call(x_sc[:half])
    n1 = sc_call(x_sc[half:])

    lw_bf16 = level_weight.astype(jnp.bfloat16)
    pw_pad = jnp.pad(position_weight, ((0, _NPAD - _NPOS), (0, 0)))

    tc_call = lambda n_mat: pl.pallas_call(
        _tc_body,
        grid=(Bh // _IBT,),
        in_specs=[
            pl.BlockSpec((_IBT * _NPAD, _NLEV), lambda i: (i, 0)),
            pl.BlockSpec((_NPAD, D), lambda i: (0, 0)),
            pl.BlockSpec((_NLEV, D), lambda i: (0, 0)),
        ],
        out_specs=pl.BlockSpec((_IBT, D), lambda i: (i, 0)),
        out_shape=jax.ShapeDtypeStruct((Bh, D), jnp.float32),
    )(n_mat, pw_pad, lw_bf16)
    return jnp.concatenate([tc_call(n0), tc_call(n1)], axis=0)


# speedup vs baseline: 1.0169x; 1.0169x over previous
import functools

import jax
import jax.numpy as jnp
from jax import lax
from jax.experimental import pallas as pl
from jax.experimental.pallas import tpu as pltpu
from jax.experimental.pallas import tpu_sc as plsc

_PATCH = 14
_NPOS = _PATCH * _PATCH
_NPOSP = 208
_NPAD = 200
_NLEV = 256
_NWORDS = _NPAD * _NLEV
_IBT = 32


def _sc_hist_body(x_hbm, n_hbm, xv, nbuf, nbuf2, sem0, sem1, imgs_per_worker, num_cores):
    wid = lax.axis_index("s") * num_cores + lax.axis_index("c")
    zeros = jnp.zeros((16,), jnp.float32)
    ones = jnp.ones((16,), jnp.float32)
    iota16 = lax.broadcasted_iota(jnp.int32, (16,), 0)

    def _clear(r, carry):
        for k in range(_NLEV // 16):
            nbuf[r, pl.ds(k * 16, 16)] = zeros
            nbuf2[r, pl.ds(k * 16, 16)] = zeros
        return carry

    lax.fori_loop(0, _NPAD, _clear, 0)

    def _indices(img, chunk):
        p, c = chunk
        xc = xv[pl.ds(img * 4 * _NPOSP + p * _NPOSP + c * 16, 16)]
        t = xc * float(_NLEV - 1) + 8388608.0
        lv = (t - 8388608.0).astype(jnp.int32)
        lv = jnp.minimum(jnp.maximum(lv, 0), _NLEV - 1)
        jv = jnp.minimum(iota16 + c * 16, _NPAD - 1)
        mask = (iota16 + c * 16) < (_NPOS if c == 12 else _NPOSP)
        return jv, lv, mask

    chunks = [(p, c) for p in range(4) for c in range(13)]
    base = wid * imgs_per_worker
    pltpu.sync_copy(x_hbm.at[pl.ds(base * 4 * _NPOSP, imgs_per_worker * 4 * _NPOSP)], xv)
    bufs = (nbuf, nbuf2)
    sems = (sem0, sem1)
    copies = [None, None]
    for img in range(imgs_per_worker):
        k = img % 2
        buf = bufs[k]
        if copies[k] is not None:
            copies[k].wait()
            for ch in chunks:
                jv, lv, mask = _indices(img - 2, ch)
                plsc.store_scatter(buf, [jv, lv], zeros, mask=mask)
        for ch in chunks:
            jv, lv, mask = _indices(img, ch)
            plsc.addupdate_scatter(buf, [jv, lv], ones, mask=mask)
        copies[k] = pltpu.async_copy(
            buf, n_hbm.at[pl.ds((base + img) * _NPAD, _NPAD)], sems[k])
    for k in range(2):
        if copies[k] is not None:
            copies[k].wait()


def _tc_body(n_ref, pw_ref, lw_ref, out_ref):
    pw = pw_ref[...]
    lw = lw_ref[...]
    for i in range(_IBT):
        cnt = n_ref[i * _NPAD:(i + 1) * _NPAD, :].astype(jnp.bfloat16)
        m = jax.lax.dot_general(
            cnt, lw, (((1,), (0,)), ((), ())),
            preferred_element_type=jnp.float32,
        )
        s = jnp.sum(m * pw, axis=0)
        out_ref[i, :] = jnp.where(s > 0.0, 1.0, -1.0)


def kernel(x, position_weight, level_weight):
    B, C, H, W = x.shape
    p = _PATCH
    D = position_weight.shape[1]
    x_pj = x.reshape(B, C, H // p, p, W // p, p)
    x_pj = x_pj.transpose(0, 1, 2, 4, 3, 5).reshape(B, 4, p * p)
    x_pj = jnp.pad(x_pj, ((0, 0), (0, 0), (0, _NPOSP - _NPOS)))
    x_sc = x_pj.reshape(B * 4 * _NPOSP)

    info = plsc.get_sparse_core_info()
    nw = info.num_cores * info.num_subcores
    sc_hist = functools.partial(
        _sc_hist_body,
        imgs_per_worker=B // nw,
        num_cores=info.num_cores,
    )
    n_flat = pl.kernel(
        sc_hist,
        out_type=jax.ShapeDtypeStruct((B * _NPAD, _NLEV), jnp.float32),
        mesh=plsc.VectorSubcoreMesh(
            core_axis_name="c", subcore_axis_name="s"),
        compiler_params=pltpu.CompilerParams(needs_layout_passes=False),
        scratch_types=[
            pltpu.VMEM((B // nw * 4 * _NPOSP,), jnp.float32),
            pltpu.VMEM((_NPAD, _NLEV), jnp.float32),
            pltpu.VMEM((_NPAD, _NLEV), jnp.float32),
            pltpu.SemaphoreType.DMA,
            pltpu.SemaphoreType.DMA,
        ],
    )(x_sc)
    n_mat = n_flat

    lw_bf16 = level_weight.astype(jnp.bfloat16)
    pw_pad = jnp.pad(position_weight, ((0, _NPAD - _NPOS), (0, 0)))

    grid = (B // _IBT,)
    return pl.pallas_call(
        _tc_body,
        grid=grid,
        in_specs=[
            pl.BlockSpec((_IBT * _NPAD, _NLEV), lambda i: (i, 0)),
            pl.BlockSpec((_NPAD, D), lambda i: (0, 0)),
            pl.BlockSpec((_NLEV, D), lambda i: (0, 0)),
        ],
        out_specs=pl.BlockSpec((_IBT, D), lambda i: (i, 0)),
        out_shape=jax.ShapeDtypeStruct((B, D), jnp.float32),
    )(n_mat, pw_pad, lw_bf16)
